# baseline (speedup 1.0000x reference)
import functools

import jax
import jax.numpy as jnp
from jax.experimental import pallas as pl
from jax.experimental.pallas import tpu as pltpu


def _linear_sigmoid_kernel(x_ref, w_ref, b_ref, o_ref, *, chunk):
    wb = w_ref[...].astype(jnp.bfloat16)
    b = b_ref[...]
    tm = x_ref.shape[0]
    for r in range(tm // chunk):
        xs = x_ref[pl.ds(r * chunk, chunk), :].astype(jnp.bfloat16)
        acc = jnp.dot(xs, wb, preferred_element_type=jnp.float32)
        o_ref[pl.ds(r * chunk, chunk), :] = jax.nn.sigmoid(acc + b)


@jax.jit
def kernel(x, w_t, b2d):
    x = x.astype(jnp.float32)
    M, K = x.shape
    K2, N = w_t.shape
    assert K == K2 and b2d.shape == (1, N)

    w_t = w_t.astype(jnp.float32)
    b2d = b2d.astype(jnp.float32)

    tm = 4096
    while M % tm != 0 and tm > 8:
        tm //= 2
    tn = 512
    while N % tn != 0 and tn > 128:
        tn //= 2
    m_pad = M
    if M % tm != 0:
        m_pad = ((M + tm - 1) // tm) * tm
        x = jnp.pad(x, ((0, m_pad - M), (0, 0)))

    chunk = 256
    while tm % chunk != 0 and chunk > 8:
        chunk //= 2

    out = pl.pallas_call(
        functools.partial(_linear_sigmoid_kernel, chunk=chunk),
        out_shape=jax.ShapeDtypeStruct((m_pad, N), jnp.float32),
        grid=(m_pad // tm, N // tn),
        in_specs=[
            pl.BlockSpec((tm, K), lambda i, j: (i, 0)),
            pl.BlockSpec((K, tn), lambda i, j: (0, j)),
            pl.BlockSpec((1, tn), lambda i, j: (0, j)),
        ],
        out_specs=pl.BlockSpec((tm, tn), lambda i, j: (i, j)),
        compiler_params=pltpu.CompilerParams(
            dimension_semantics=("arbitrary", "arbitrary"),
        ),
    )(x, w_t, b2d)

    if m_pad != M:
        out = out[:M]
    return out


# speedup vs baseline: 1.2517x; 1.2517x over previous
import functools

import jax
import jax.numpy as jnp
from jax.experimental import pallas as pl
from jax.experimental.pallas import tpu as pltpu


def _linear_sigmoid_pipeline(x_hbm, w_ref, b_ref, o_hbm, xbuf, obuf, rsem,
                             wsem, *, nslabs, slab, chunk):
    wb = w_ref[...].astype(jnp.bfloat16)
    b = b_ref[...]

    def read(s, slot):
        return pltpu.make_async_copy(
            x_hbm.at[pl.ds(s * slab, slab), :], xbuf.at[slot], rsem.at[slot])

    def write(s, slot):
        return pltpu.make_async_copy(
            obuf.at[slot], o_hbm.at[pl.ds(s * slab, slab), :], wsem.at[slot])

    read(0, 0).start()
    for s in range(nslabs):
        slot = s % 2
        if s + 1 < nslabs:
            read(s + 1, (s + 1) % 2).start()
        read(s, slot).wait()
        if s >= 2:
            write(s - 2, slot).wait()
        for r in range(slab // chunk):
            xs = xbuf[slot, pl.ds(r * chunk, chunk), :].astype(jnp.bfloat16)
            acc = jnp.dot(xs, wb, preferred_element_type=jnp.float32)
            obuf[slot, pl.ds(r * chunk, chunk), :] = jax.nn.sigmoid(acc + b)
        write(s, slot).start()
    for s in range(max(0, nslabs - 2), nslabs):
        write(s, s % 2).wait()


@jax.jit
def kernel(x, w_t, b2d):
    x = x.astype(jnp.float32)
    M, K = x.shape
    K2, N = w_t.shape
    assert K == K2 and b2d.shape == (1, N)

    w_t = w_t.astype(jnp.float32)
    b2d = b2d.astype(jnp.float32)

    slab = 2048
    while M % slab != 0 and slab > 8:
        slab //= 2
    m_pad = M
    if M % slab != 0:
        m_pad = ((M + slab - 1) // slab) * slab
        x = jnp.pad(x, ((0, m_pad - M), (0, 0)))
    nslabs = m_pad // slab

    chunk = 256
    while slab % chunk != 0 and chunk > 8:
        chunk //= 2

    out = pl.pallas_call(
        functools.partial(_linear_sigmoid_pipeline,
                          nslabs=nslabs, slab=slab, chunk=chunk),
        out_shape=jax.ShapeDtypeStruct((m_pad, N), jnp.float32),
        in_specs=[
            pl.BlockSpec(memory_space=pl.ANY),
            pl.BlockSpec((K, N), lambda: (0, 0)),
            pl.BlockSpec((1, N), lambda: (0, 0)),
        ],
        out_specs=pl.BlockSpec(memory_space=pl.ANY),
        scratch_shapes=[
            pltpu.VMEM((2, slab, K), jnp.float32),
            pltpu.VMEM((2, slab, N), jnp.float32),
            pltpu.SemaphoreType.DMA((2,)),
            pltpu.SemaphoreType.DMA((2,)),
        ],
    )(x, w_t, b2d)

    if m_pad != M:
        out = out[:M]
    return out


# speedup vs baseline: 1.3021x; 1.0403x over previous
import functools

import jax
import jax.numpy as jnp
from jax.experimental import pallas as pl
from jax.experimental.pallas import tpu as pltpu


def _linear_sigmoid_pipeline(x_hbm, w_ref, b_ref, o_hbm, xbuf, obuf, rsem,
                             wsem, *, nslabs, slab, chunk, nbuf):
    wb = w_ref[...].astype(jnp.bfloat16)
    b = b_ref[...]

    def read(s):
        slot = s % nbuf
        return pltpu.make_async_copy(
            x_hbm.at[pl.ds(s * slab, slab), :], xbuf.at[slot], rsem.at[slot])

    def write(s):
        slot = s % nbuf
        return pltpu.make_async_copy(
            obuf.at[slot], o_hbm.at[pl.ds(s * slab, slab), :], wsem.at[slot])

    for s in range(min(nbuf, nslabs)):
        read(s).start()
    for s in range(nslabs):
        slot = s % nbuf
        read(s).wait()
        if s >= nbuf:
            write(s - nbuf).wait()
        for r in range(slab // chunk):
            xs = xbuf[slot, pl.ds(r * chunk, chunk), :].astype(jnp.bfloat16)
            acc = jnp.dot(xs, wb, preferred_element_type=jnp.float32)
            obuf[slot, pl.ds(r * chunk, chunk), :] = jax.nn.sigmoid(acc + b)
        write(s).start()
        if s + nbuf < nslabs:
            read(s + nbuf).start()
    for s in range(max(0, nslabs - nbuf), nslabs):
        write(s).wait()


@jax.jit
def kernel(x, w_t, b2d):
    x = x.astype(jnp.float32)
    M, K = x.shape
    K2, N = w_t.shape
    assert K == K2 and b2d.shape == (1, N)

    w_t = w_t.astype(jnp.float32)
    b2d = b2d.astype(jnp.float32)

    slab = 1024
    nbuf = 4
    while M % slab != 0 and slab > 8:
        slab //= 2
    m_pad = M
    if M % slab != 0:
        m_pad = ((M + slab - 1) // slab) * slab
        x = jnp.pad(x, ((0, m_pad - M), (0, 0)))
    nslabs = m_pad // slab

    chunk = 256
    while slab % chunk != 0 and chunk > 8:
        chunk //= 2

    out = pl.pallas_call(
        functools.partial(_linear_sigmoid_pipeline,
                          nslabs=nslabs, slab=slab, chunk=chunk, nbuf=nbuf),
        out_shape=jax.ShapeDtypeStruct((m_pad, N), jnp.float32),
        in_specs=[
            pl.BlockSpec(memory_space=pl.ANY),
            pl.BlockSpec((K, N), lambda: (0, 0)),
            pl.BlockSpec((1, N), lambda: (0, 0)),
        ],
        out_specs=pl.BlockSpec(memory_space=pl.ANY),
        scratch_shapes=[
            pltpu.VMEM((nbuf, slab, K), jnp.float32),
            pltpu.VMEM((nbuf, slab, N), jnp.float32),
            pltpu.SemaphoreType.DMA((nbuf,)),
            pltpu.SemaphoreType.DMA((nbuf,)),
        ],
    )(x, w_t, b2d)

    if m_pad != M:
        out = out[:M]
    return out
